# baseline (speedup 1.0000x reference)
import functools
import math

import jax
import jax.numpy as jnp
from jax import lax
from jax.experimental import pallas as pl
from jax.experimental.pallas import tpu as pltpu
from jax.experimental.pallas import tpu_sc as plsc

D = 64
CHUNK = 128
SCALE = math.sqrt(D)


@functools.cache
def _make_kernel(N):
    info = plsc.get_sparse_core_info()
    NC, NS, L = info.num_cores, info.num_subcores, info.num_lanes
    NW = NC * NS
    per_w = N // NW
    steps = per_w // CHUNK
    assert per_w % CHUNK == 0 and N % NW == 0
    mesh = plsc.VectorSubcoreMesh(core_axis_name="c", subcore_axis_name="s")

    @functools.partial(
        pl.kernel,
        mesh=mesh,
        compiler_params=pltpu.CompilerParams(use_tc_tiling_on_sc=False),
        out_type=jax.ShapeDtypeStruct((N, D), jnp.float32),
        scratch_types=[
            pltpu.VMEM((steps, CHUNK), jnp.int32),
            pltpu.VMEM((CHUNK, D), jnp.float32),
            pltpu.SemaphoreType.DMA,
        ],
    )
    def k(x_hbm, table_hbm, out_hbm, idx_v, rows_v, sem):
        wid = lax.axis_index("s") * NC + lax.axis_index("c")
        row_base = wid * steps
        out_base = wid * per_w
        pltpu.sync_copy(x_hbm.at[pl.ds(row_base, steps)], idx_v)

        def step(j, _):
            pltpu.async_copy(table_hbm.at[idx_v.at[j]], rows_v, sem).wait()

            def scale_row(r, _2):
                for t in range(D // 16):
                    rows_v[r, pl.ds(t * 16, 16)] = (
                        rows_v[r, pl.ds(t * 16, 16)] * SCALE
                    )
                return 0

            lax.fori_loop(0, CHUNK, scale_row, 0)
            pltpu.sync_copy(rows_v, out_hbm.at[pl.ds(out_base + j * CHUNK, CHUNK)])
            return 0

        lax.fori_loop(0, steps, step, 0)

    return k


def kernel(x, table):
    B, L = x.shape
    N = B * L
    xf = x.reshape(N // CHUNK, CHUNK).astype(jnp.int32)
    out = _make_kernel(N)(xf, table)
    return out.reshape(B, L, D)


# speedup vs baseline: 1.1888x; 1.1888x over previous
import functools
import math

import jax
import jax.numpy as jnp
from jax import lax
from jax.experimental import pallas as pl
from jax.experimental.pallas import tpu as pltpu
from jax.experimental.pallas import tpu_sc as plsc

D = 64
CHUNK = 128
SUB = 2
ROWS = SUB * CHUNK
SCALE = math.sqrt(D)
UNROLL = 4


@functools.cache
def _make_kernel(N):
    info = plsc.get_sparse_core_info()
    NC, NS = info.num_cores, info.num_subcores
    NW = NC * NS
    per_w = N // NW
    steps = per_w // ROWS
    assert N % NW == 0 and per_w % ROWS == 0 and steps % 2 == 0
    mesh = plsc.VectorSubcoreMesh(core_axis_name="c", subcore_axis_name="s")

    @functools.partial(
        pl.kernel,
        mesh=mesh,
        compiler_params=pltpu.CompilerParams(use_tc_tiling_on_sc=False),
        out_type=jax.ShapeDtypeStruct((N, D), jnp.float32),
        scratch_types=[
            pltpu.VMEM((steps * SUB, CHUNK), jnp.int32),
            pltpu.VMEM((ROWS, D), jnp.float32),
            pltpu.VMEM((ROWS, D), jnp.float32),
            pltpu.SemaphoreType.DMA,
            pltpu.SemaphoreType.DMA,
            pltpu.SemaphoreType.DMA,
            pltpu.SemaphoreType.DMA,
        ],
    )
    def k(x_hbm, table_hbm, out_hbm, idx_v, buf0, buf1, gsem0, gsem1, ssem0, ssem1):
        wid = lax.axis_index("s") * NC + lax.axis_index("c")
        out_base = wid * per_w
        pltpu.sync_copy(x_hbm.at[pl.ds(wid * steps * SUB, steps * SUB)], idx_v)

        def fire_gather(j, buf, gsem):
            for s in range(SUB):
                pltpu.async_copy(
                    table_hbm.at[idx_v.at[j * SUB + s]],
                    buf.at[pl.ds(s * CHUNK, CHUNK)],
                    gsem,
                )

        def drain_gather(j, buf, gsem):
            for s in range(SUB):
                pltpu.make_async_copy(
                    table_hbm.at[idx_v.at[j * SUB + s]],
                    buf.at[pl.ds(s * CHUNK, CHUNK)],
                    gsem,
                ).wait()

        def fire_store(j, buf, ssem):
            pltpu.async_copy(buf, out_hbm.at[pl.ds(out_base + j * ROWS, ROWS)], ssem)

        def drain_store(buf, ssem):
            pltpu.make_async_copy(buf, out_hbm.at[pl.ds(out_base, ROWS)], ssem).wait()

        def scale(buf):
            def body(r, _):
                for u in range(UNROLL):
                    for t in range(D // 16):
                        sl = (r * UNROLL + u, pl.ds(t * 16, 16))
                        buf[sl] = buf[sl] * SCALE
                return 0

            lax.fori_loop(0, ROWS // UNROLL, body, 0)

        def halfstep(j, first, buf, gsem, obuf, ogsem, ossem, ssem):
            drain_gather(j, buf, gsem)
            if first:
                @pl.when(j >= 1)
                def _():
                    drain_store(obuf, ossem)
            else:
                drain_store(obuf, ossem)

            @pl.when(j + 1 < steps)
            def _():
                fire_gather(j + 1, obuf, ogsem)

            scale(buf)
            fire_store(j, buf, ssem)

        fire_gather(0, buf0, gsem0)

        def body(t, _):
            halfstep(2 * t, True, buf0, gsem0, buf1, gsem1, ssem1, ssem0)
            halfstep(2 * t + 1, False, buf1, gsem1, buf0, gsem0, ssem0, ssem1)
            return 0

        lax.fori_loop(0, steps // 2, body, 0)
        drain_store(buf1, ssem1)

    return k


def kernel(x, table):
    B, L = x.shape
    N = B * L
    xf = x.reshape(N // CHUNK, CHUNK).astype(jnp.int32)
    out = _make_kernel(N)(xf, table)
    return out.reshape(B, L, D)
